# baseline (speedup 1.0000x reference)
import jax
import jax.numpy as jnp
from jax.experimental import pallas as pl
from jax.experimental.pallas import tpu as pltpu

N = 512
D = 128
H = 128
T = 16
B = 4
K = 3
C = 4
F32 = jnp.float32


def _dot(a, b):
    return jnp.dot(a.astype(jnp.bfloat16), b.astype(jnp.bfloat16),
                   preferred_element_type=F32)


def _input_contrib(x0, s, w_in, bias):
    x1 = _dot(s, x0)
    x2 = 2.0 * _dot(s, x1) - x0
    a_list = []
    for bi in range(B):
        sl = slice(bi * D, (bi + 1) * D)
        xc = jnp.concatenate([x0[:, sl], x1[:, sl], x2[:, sl]], axis=1)
        a_list.append(_dot(xc, w_in) + bias)
    return a_list


def _gru_step(a_list, s, wg, wc, h0, rs_ref):
    h1 = _dot(s, h0)
    h2 = 2.0 * _dot(s, h1) - h0
    us = []
    for bi in range(B):
        sl = slice(bi * H, (bi + 1) * H)
        xc = jnp.concatenate([h0[:, sl], h1[:, sl], h2[:, sl]], axis=1)
        g = jax.nn.sigmoid(a_list[bi][:, : 2 * H] + _dot(xc, wg))
        r, u = g[:, :H], g[:, H:]
        rs_ref[:, sl] = r * h0[:, sl]
        us.append(u)
    rs0 = rs_ref[...]
    rs1 = _dot(s, rs0)
    rs2 = 2.0 * _dot(s, rs1) - rs0
    new_states = []
    for bi in range(B):
        sl = slice(bi * H, (bi + 1) * H)
        xc = jnp.concatenate([rs0[:, sl], rs1[:, sl], rs2[:, sl]], axis=1)
        c = jnp.tanh(a_list[bi][:, 2 * H:] + _dot(xc, wc))
        u = us[bi]
        new_states.append(u * h0[:, sl] + (1.0 - u) * c)
    return new_states


def _mega_body(x_ref, s_ref, w0in_ref, b0_ref, wg0_ref, wc0_ref,
               w1in_ref, b1_ref, wg1_ref, wc1_ref, m_ref, wfc_ref, bfc_ref,
               o_ref, st0_ref, rs0_ref, pipe_ref, st1_ref, rs1_ref, last_ref):
    t = pl.program_id(0)

    @pl.when(t == 0)
    def _():
        st0_ref[...] = jnp.zeros_like(st0_ref)
        st1_ref[...] = jnp.zeros_like(st1_ref)
        pipe_ref[...] = jnp.zeros_like(pipe_ref)
        last_ref[...] = jnp.zeros_like(last_ref)

    s = s_ref[...]
    live1 = jnp.where(t > 0, 1.0, 0.0).astype(F32)

    a1 = _input_contrib(pipe_ref[...], s, w1in_ref[...], b1_ref[0])
    ns1 = _gru_step(a1, s, wg1_ref[...], wc1_ref[...], st1_ref[...], rs1_ref)
    for bi in range(B):
        sl = slice(bi * H, (bi + 1) * H)
        st1_ref[:, sl] = live1 * ns1[bi]
        mb = live1 * m_ref[0, 0, bi]
        last_ref[:, sl] = mb * ns1[bi] + (1.0 - mb) * last_ref[:, sl]

    a0 = _input_contrib(x_ref[0], s, w0in_ref[...], b0_ref[0])
    ns0 = _gru_step(a0, s, wg0_ref[...], wc0_ref[...], st0_ref[...], rs0_ref)
    for bi in range(B):
        sl = slice(bi * H, (bi + 1) * H)
        st0_ref[:, sl] = ns0[bi]
        pipe_ref[:, sl] = ns0[bi]

    @pl.when(t == T)
    def _():
        wfc = wfc_ref[...]
        bfc = bfc_ref[0]
        for bi in range(B):
            sl = slice(bi * H, (bi + 1) * H)
            lg = _dot(jnp.maximum(last_ref[:, sl], 0.0), wfc) + bfc
            o_ref[bi:bi + 1, :] = jnp.max(lg, axis=0, keepdims=True)


def _mega(x, s, w0_in, bias0, wg0_h, wc0_h, w1_in, bias1, wg1_h, wc1_h,
          mask, wfc_pad, bfc_pad):
    return pl.pallas_call(
        _mega_body,
        grid=(T + 1,),
        in_specs=[
            pl.BlockSpec((1, N, B * D), lambda t: (jnp.minimum(t, T - 1), 0, 0)),
            pl.BlockSpec((N, N), lambda t: (0, 0)),
            pl.BlockSpec((K * D, 3 * H), lambda t: (0, 0)),
            pl.BlockSpec((1, 3 * H), lambda t: (0, 0)),
            pl.BlockSpec((K * H, 2 * H), lambda t: (0, 0)),
            pl.BlockSpec((K * H, H), lambda t: (0, 0)),
            pl.BlockSpec((K * H, 3 * H), lambda t: (0, 0)),
            pl.BlockSpec((1, 3 * H), lambda t: (0, 0)),
            pl.BlockSpec((K * H, 2 * H), lambda t: (0, 0)),
            pl.BlockSpec((K * H, H), lambda t: (0, 0)),
            pl.BlockSpec((1, 1, B), lambda t: (jnp.maximum(t - 1, 0), 0, 0)),
            pl.BlockSpec((H, 128), lambda t: (0, 0)),
            pl.BlockSpec((1, 128), lambda t: (0, 0)),
        ],
        out_specs=pl.BlockSpec((B, 128), lambda t: (0, 0)),
        out_shape=jax.ShapeDtypeStruct((B, 128), F32),
        scratch_shapes=[
            pltpu.VMEM((N, B * H), F32),
            pltpu.VMEM((N, B * H), F32),
            pltpu.VMEM((N, B * H), F32),
            pltpu.VMEM((N, B * H), F32),
            pltpu.VMEM((N, B * H), F32),
            pltpu.VMEM((N, B * H), F32),
        ],
    )(x, s, w0_in, bias0, wg0_h, wc0_h, w1_in, bias1, wg1_h, wc1_h,
      mask, wfc_pad, bfc_pad)


def _split_weight(w, din, dout):
    wr = w.reshape(din + H, K, dout)
    w_in = wr[:din].transpose(1, 0, 2).reshape(K * din, dout)
    w_h = wr[din:].transpose(1, 0, 2).reshape(K * H, dout)
    return w_in, w_h


def kernel(input_seq, seq_lengths, supports, Wg0, bg0, Wc0, bc0,
           Wg1, bg1, Wc1, bc1, Wfc, bfc):
    s = supports[0]

    wg0_in, wg0_h = _split_weight(Wg0, D, 2 * H)
    wc0_in, wc0_h = _split_weight(Wc0, D, H)
    wg1_in, wg1_h = _split_weight(Wg1, H, 2 * H)
    wc1_in, wc1_h = _split_weight(Wc1, H, H)
    w0_in = jnp.concatenate([wg0_in, wc0_in], axis=1)
    w1_in = jnp.concatenate([wg1_in, wc1_in], axis=1)
    bias0 = jnp.concatenate([bg0, bc0]).reshape(1, 3 * H)
    bias1 = jnp.concatenate([bg1, bc1]).reshape(1, 3 * H)

    idx = jnp.clip(seq_lengths - 1, 0, T - 1).astype(jnp.int32)
    mask = (jnp.arange(T, dtype=jnp.int32)[:, None, None]
            == idx[None, None, :]).astype(F32)

    wfc_pad = jnp.zeros((H, 128), F32).at[:, :C].set(Wfc)
    bfc_pad = jnp.zeros((1, 128), F32).at[0, :C].set(bfc)

    x0 = input_seq.transpose(1, 2, 0, 3).reshape(T, N, B * D)
    logits_pad = _mega(x0, s, w0_in, bias0, wg0_h, wc0_h,
                       w1_in, bias1, wg1_h, wc1_h, mask, wfc_pad, bfc_pad)
    return logits_pad[:, :C]
